# baseline (speedup 1.0000x reference)
import functools

import jax
import jax.numpy as jnp
from jax import lax
from jax.experimental import pallas as pl
from jax.experimental.pallas import tpu as pltpu
from jax.experimental.pallas import tpu_sc as plsc

DB = 32
NS = 16
LANES = 16
CHUNK = 512


def _vbroadcast(vec, lane):
    idx = jnp.full((LANES, 1), lane, jnp.int32)
    return lax.gather(
        vec, idx,
        lax.GatherDimensionNumbers(offset_dims=(), collapsed_slice_dims=(0,),
                                   start_index_map=(0,)),
        (1,), mode=lax.GatherScatterMode.PROMISE_IN_BOUNDS)


def _proj_body(x_ref, w_ref, xo_ref, xpn_ref):
    cat = jnp.dot(x_ref[...], w_ref[...], preferred_element_type=jnp.float32)
    xo_ref[...] = cat[:, 0:DB]
    xpn_ref[0] = cat[:, DB:2 * DB]
    xpn_ref[1] = cat[:, 2 * DB:3 * DB]


def _post_body(xo_ref, sp_ref, sn_ref, wm_ref, g_ref, b_ref, a_ref, o_ref):
    n = xo_ref.shape[0]
    a = a_ref[0]

    def bn_prelu(v, j):
        g = g_ref[j, :]
        b = b_ref[j, :]
        mean = jnp.sum(v, axis=0, keepdims=True) * (1.0 / n)
        var = jnp.sum(v * v, axis=0, keepdims=True) * (1.0 / n) - mean * mean
        y = g * (v - mean) * jax.lax.rsqrt(var + 1e-5) + b
        return jnp.where(y >= 0, y, a * y)

    cat = jnp.concatenate(
        [bn_prelu(xo_ref[...], 0),
         bn_prelu(sp_ref[...], 1),
         bn_prelu(sn_ref[...], 2)], axis=1)
    e = jnp.dot(cat, wm_ref[...], preferred_element_type=jnp.float32)
    nrm = jnp.sqrt(jnp.sum(e * e, axis=1, keepdims=True))
    o_ref[...] = e / jnp.maximum(nrm, 1e-12)


GR = 3
IR = 4


def _spmm_sc_body(n, ep, xpn_hbm, pidx_hbm, pw_hbm, nidx_hbm, nw_hbm, z_hbm,
                  out_hbm, col_v, row_v, w_v, gath_v, acc_s, xs_s,
                  gsem, ssem, isem_a, isem_b):
    c = lax.axis_index("c")
    s = lax.axis_index("s")

    zr = n // NS
    pltpu.sync_copy(z_hbm.at[pl.ds(s * zr, zr)], acc_s.at[pl.ds(s * zr, zr)])
    pltpu.sync_copy(xpn_hbm.at[c, pl.ds(s * zr, zr)],
                    xs_s.at[pl.ds(s * zr, zr)])
    plsc.subcore_barrier()

    tile_edges = ep // NS
    n_chunks = tile_edges // CHUNK

    def fire_idx(k, sem):
        m = k % IR
        eoff = s * tile_edges + k * CHUNK

        @pl.when(c == 0)
        def _():
            pltpu.async_copy(pidx_hbm.at[1, pl.ds(eoff, CHUNK)],
                             col_v.at[m], sem)
            pltpu.async_copy(pidx_hbm.at[0, pl.ds(eoff, CHUNK)],
                             row_v.at[m], sem)
            pltpu.async_copy(pw_hbm.at[pl.ds(eoff, CHUNK)], w_v.at[m], sem)

        @pl.when(c == 1)
        def _():
            pltpu.async_copy(nidx_hbm.at[1, pl.ds(eoff, CHUNK)],
                             col_v.at[m], sem)
            pltpu.async_copy(nidx_hbm.at[0, pl.ds(eoff, CHUNK)],
                             row_v.at[m], sem)
            pltpu.async_copy(nw_hbm.at[pl.ds(eoff, CHUNK)], w_v.at[m], sem)

    def drain_idx(k, sem):
        m = k % IR
        pltpu.make_async_copy(pidx_hbm.at[1, pl.ds(0, CHUNK)],
                              col_v.at[m], sem).wait()
        pltpu.make_async_copy(pidx_hbm.at[0, pl.ds(0, CHUNK)],
                              row_v.at[m], sem).wait()
        pltpu.make_async_copy(pw_hbm.at[pl.ds(0, CHUNK)], w_v.at[m],
                              sem).wait()

    def fire_idx_alt(k):
        @pl.when(k % 2 == 0)
        def _():
            fire_idx(k, isem_a)

        @pl.when(k % 2 == 1)
        def _():
            fire_idx(k, isem_b)

    def drain_idx_alt(k):
        @pl.when(k % 2 == 0)
        def _():
            drain_idx(k, isem_a)

        @pl.when(k % 2 == 1)
        def _():
            drain_idx(k, isem_b)

    def fire_gather(k):
        pltpu.async_copy(xs_s.at[col_v.at[k % IR]], gath_v.at[k % GR], gsem)

    def drain_gather(k):
        pltpu.make_async_copy(xs_s.at[col_v.at[k % IR]], gath_v.at[k % GR],
                              gsem).wait()

    def fire_scatter(k):
        pltpu.async_copy(gath_v.at[k % GR], acc_s.at[row_v.at[k % IR]],
                         ssem, add=True)

    def drain_scatter(k):
        pltpu.make_async_copy(gath_v.at[k % GR], acc_s.at[row_v.at[k % IR]],
                              ssem).wait()

    fire_idx(0, isem_a)
    drain_idx(0, isem_a)
    fire_idx(1, isem_b)
    fire_gather(0)

    def chunk_body(k, carry):
        @pl.when(k >= 2)
        def _():
            drain_scatter(k - 2)

        @pl.when(k < n_chunks - 2)
        def _():
            fire_idx_alt(k + 2)

        drain_gather(k)

        @pl.when(k < n_chunks - 1)
        def _():
            drain_idx_alt(k + 1)
            fire_gather(k + 1)

        mg = k % GR
        mi = k % IR

        def scale_body(g, carry2):
            wgrp = w_v[mi, pl.ds(g * LANES, LANES)]
            for e in range(LANES):
                ws = _vbroadcast(wgrp, e)
                r = g * LANES + e
                gath_v[mg, r, 0:16] = gath_v[mg, r, 0:16] * ws
                gath_v[mg, r, 16:32] = gath_v[mg, r, 16:32] * ws
            return carry2

        lax.fori_loop(0, CHUNK // LANES, scale_body, 0)

        fire_scatter(k)
        return carry

    lax.fori_loop(0, n_chunks, chunk_body, 0)
    drain_scatter(n_chunks - 2)
    drain_scatter(n_chunks - 1)
    plsc.subcore_barrier()

    pltpu.sync_copy(acc_s.at[pl.ds(s * zr, zr)],
                    out_hbm.at[c, pl.ds(s * zr, zr)])


def kernel(x, pos_index, pos_weight, neg_index, neg_weight, other_index,
           other_weight, W_org, W_pos, W_neg, W_mlp, g_org, b_org, g_pos,
           b_pos, g_neg, b_neg, prelu_a):
    n, da = x.shape
    e = pos_index.shape[1]
    f32 = jnp.float32

    wcat = jnp.concatenate([W_org, W_pos, W_neg], axis=1)
    xo, xpn = pl.pallas_call(
        _proj_body,
        out_shape=(jax.ShapeDtypeStruct((n, DB), f32),
                   jax.ShapeDtypeStruct((2, n, DB), f32)),
    )(x, wcat)

    step = NS * CHUNK
    ep = ((e + step - 1) // step) * step
    pad = ep - e
    if pad:
        pos_index = jnp.pad(pos_index, ((0, 0), (0, pad)))
        neg_index = jnp.pad(neg_index, ((0, 0), (0, pad)))
        pos_weight = jnp.pad(pos_weight, (0, pad))
        neg_weight = jnp.pad(neg_weight, (0, pad))
    zeros = jnp.zeros((n, DB), f32)

    mesh = plsc.VectorSubcoreMesh(core_axis_name="c", subcore_axis_name="s")
    spmm = pl.kernel(
        functools.partial(_spmm_sc_body, n, ep),
        out_type=jax.ShapeDtypeStruct((2, n, DB), f32),
        mesh=mesh,
        scratch_types=[
            pltpu.VMEM((IR, CHUNK), jnp.int32),
            pltpu.VMEM((IR, CHUNK), jnp.int32),
            pltpu.VMEM((IR, CHUNK), f32),
            pltpu.VMEM((GR, CHUNK, DB), f32),
            pltpu.VMEM_SHARED((n, DB), f32),
            pltpu.VMEM_SHARED((n, DB), f32),
            pltpu.SemaphoreType.DMA,
            pltpu.SemaphoreType.DMA,
            pltpu.SemaphoreType.DMA,
            pltpu.SemaphoreType.DMA,
        ],
        compiler_params=pltpu.CompilerParams(use_tc_tiling_on_sc=False),
    )
    seg = spmm(xpn, pos_index, pos_weight, neg_index, neg_weight, zeros)

    gs = jnp.stack([g_org, g_pos, g_neg]).reshape(3, DB)
    bs = jnp.stack([b_org, b_pos, b_neg]).reshape(3, DB)
    embs = pl.pallas_call(
        _post_body,
        out_shape=jax.ShapeDtypeStruct((n, DB), f32),
        in_specs=[
            pl.BlockSpec(memory_space=pltpu.VMEM),
            pl.BlockSpec(memory_space=pltpu.VMEM),
            pl.BlockSpec(memory_space=pltpu.VMEM),
            pl.BlockSpec(memory_space=pltpu.VMEM),
            pl.BlockSpec(memory_space=pltpu.VMEM),
            pl.BlockSpec(memory_space=pltpu.VMEM),
            pl.BlockSpec(memory_space=pltpu.SMEM),
        ],
    )(xo, seg[0], seg[1], W_mlp, gs, bs, prelu_a.reshape(1))
    return embs


# speedup vs baseline: 1.6608x; 1.6608x over previous
import functools

import jax
import jax.numpy as jnp
from jax import lax
from jax.experimental import pallas as pl
from jax.experimental.pallas import tpu as pltpu
from jax.experimental.pallas import tpu_sc as plsc

DB = 32
NS = 16
LANES = 16
CHUNK = 1024


def _vbroadcast(vec, lane):
    idx = jnp.full((LANES, 1), lane, jnp.int32)
    return lax.gather(
        vec, idx,
        lax.GatherDimensionNumbers(offset_dims=(), collapsed_slice_dims=(0,),
                                   start_index_map=(0,)),
        (1,), mode=lax.GatherScatterMode.PROMISE_IN_BOUNDS)


def _proj_body(x_ref, w_ref, xo_ref, xpn_ref):
    cat = jnp.dot(x_ref[...], w_ref[...], preferred_element_type=jnp.float32)
    xo_ref[...] = cat[:, 0:DB]
    xpn_ref[0] = cat[:, DB:2 * DB]
    xpn_ref[1] = cat[:, 2 * DB:3 * DB]


def _post_body(xo_ref, sp_ref, sn_ref, wm_ref, g_ref, b_ref, a_ref, o_ref):
    n = xo_ref.shape[0]
    a = a_ref[0]

    def bn_prelu(v, j):
        g = g_ref[j, :]
        b = b_ref[j, :]
        mean = jnp.sum(v, axis=0, keepdims=True) * (1.0 / n)
        var = jnp.sum(v * v, axis=0, keepdims=True) * (1.0 / n) - mean * mean
        y = g * (v - mean) * jax.lax.rsqrt(var + 1e-5) + b
        return jnp.where(y >= 0, y, a * y)

    cat = jnp.concatenate(
        [bn_prelu(xo_ref[...], 0),
         bn_prelu(sp_ref[...], 1),
         bn_prelu(sn_ref[...], 2)], axis=1)
    e = jnp.dot(cat, wm_ref[...], preferred_element_type=jnp.float32)
    nrm = jnp.sqrt(jnp.sum(e * e, axis=1, keepdims=True))
    o_ref[...] = e / jnp.maximum(nrm, 1e-12)


GR = 2
IR = 4


def _spmm_sc_body(n, ep, xpn_hbm, pidx_hbm, pw_hbm, nidx_hbm, nw_hbm, z_hbm,
                  out_hbm, col_v, row_v, w_v, gath_v, acc_s, xs_s,
                  gsem, ssem, isem_a, isem_b):
    c = lax.axis_index("c")
    s = lax.axis_index("s")

    zr = n // NS
    pltpu.sync_copy(z_hbm.at[pl.ds(s * zr, zr)], acc_s.at[pl.ds(s * zr, zr)])
    pltpu.sync_copy(xpn_hbm.at[c, pl.ds(s * zr, zr)],
                    xs_s.at[pl.ds(s * zr, zr)])
    plsc.subcore_barrier()

    tile_edges = ep // NS
    n_chunks = tile_edges // CHUNK

    def fire_idx(k, sem):
        m = k % IR
        eoff = s * tile_edges + k * CHUNK

        @pl.when(c == 0)
        def _():
            pltpu.async_copy(pidx_hbm.at[1, pl.ds(eoff, CHUNK)],
                             col_v.at[m], sem)
            pltpu.async_copy(pidx_hbm.at[0, pl.ds(eoff, CHUNK)],
                             row_v.at[m], sem)
            pltpu.async_copy(pw_hbm.at[pl.ds(eoff, CHUNK)], w_v.at[m], sem)

        @pl.when(c == 1)
        def _():
            pltpu.async_copy(nidx_hbm.at[1, pl.ds(eoff, CHUNK)],
                             col_v.at[m], sem)
            pltpu.async_copy(nidx_hbm.at[0, pl.ds(eoff, CHUNK)],
                             row_v.at[m], sem)
            pltpu.async_copy(nw_hbm.at[pl.ds(eoff, CHUNK)], w_v.at[m], sem)

    def drain_idx(k, sem):
        m = k % IR
        pltpu.make_async_copy(pidx_hbm.at[1, pl.ds(0, CHUNK)],
                              col_v.at[m], sem).wait()
        pltpu.make_async_copy(pidx_hbm.at[0, pl.ds(0, CHUNK)],
                              row_v.at[m], sem).wait()
        pltpu.make_async_copy(pw_hbm.at[pl.ds(0, CHUNK)], w_v.at[m],
                              sem).wait()

    def fire_idx_alt(k):
        @pl.when(k % 2 == 0)
        def _():
            fire_idx(k, isem_a)

        @pl.when(k % 2 == 1)
        def _():
            fire_idx(k, isem_b)

    def drain_idx_alt(k):
        @pl.when(k % 2 == 0)
        def _():
            drain_idx(k, isem_a)

        @pl.when(k % 2 == 1)
        def _():
            drain_idx(k, isem_b)

    def fire_gather(k):
        pltpu.async_copy(xs_s.at[col_v.at[k % IR]], gath_v.at[k % GR], gsem)

    def drain_gather(k):
        pltpu.make_async_copy(xs_s.at[col_v.at[k % IR]], gath_v.at[k % GR],
                              gsem).wait()

    def fire_scatter(k):
        pltpu.async_copy(gath_v.at[k % GR], acc_s.at[row_v.at[k % IR]],
                         ssem, add=True)

    def drain_scatter(k):
        pltpu.make_async_copy(gath_v.at[k % GR], acc_s.at[row_v.at[k % IR]],
                              ssem).wait()

    fire_idx(0, isem_a)
    drain_idx(0, isem_a)
    fire_idx(1, isem_b)
    fire_gather(0)

    def chunk_body(k, carry):
        @pl.when(k < n_chunks - 2)
        def _():
            fire_idx_alt(k + 2)

        drain_gather(k)

        @pl.when(k >= 1)
        def _():
            drain_scatter(k - 1)

        @pl.when(k < n_chunks - 1)
        def _():
            drain_idx_alt(k + 1)
            fire_gather(k + 1)

        mg = k % GR
        mi = k % IR

        def scale_body(g, carry2):
            wgrp = w_v[mi, pl.ds(g * LANES, LANES)]
            for e in range(LANES):
                ws = _vbroadcast(wgrp, e)
                r = g * LANES + e
                gath_v[mg, r, 0:16] = gath_v[mg, r, 0:16] * ws
                gath_v[mg, r, 16:32] = gath_v[mg, r, 16:32] * ws
            return carry2

        lax.fori_loop(0, CHUNK // LANES, scale_body, 0)

        fire_scatter(k)
        return carry

    lax.fori_loop(0, n_chunks, chunk_body, 0)
    drain_scatter(n_chunks - 1)
    plsc.subcore_barrier()

    pltpu.sync_copy(acc_s.at[pl.ds(s * zr, zr)],
                    out_hbm.at[c, pl.ds(s * zr, zr)])


def kernel(x, pos_index, pos_weight, neg_index, neg_weight, other_index,
           other_weight, W_org, W_pos, W_neg, W_mlp, g_org, b_org, g_pos,
           b_pos, g_neg, b_neg, prelu_a):
    n, da = x.shape
    e = pos_index.shape[1]
    f32 = jnp.float32

    wcat = jnp.concatenate([W_org, W_pos, W_neg], axis=1)
    xo, xpn = pl.pallas_call(
        _proj_body,
        out_shape=(jax.ShapeDtypeStruct((n, DB), f32),
                   jax.ShapeDtypeStruct((2, n, DB), f32)),
    )(x, wcat)

    step = NS * CHUNK
    ep = ((e + step - 1) // step) * step
    pad = ep - e
    if pad:
        pos_index = jnp.pad(pos_index, ((0, 0), (0, pad)))
        neg_index = jnp.pad(neg_index, ((0, 0), (0, pad)))
        pos_weight = jnp.pad(pos_weight, (0, pad))
        neg_weight = jnp.pad(neg_weight, (0, pad))
    zeros = jnp.zeros((n, DB), f32)

    mesh = plsc.VectorSubcoreMesh(core_axis_name="c", subcore_axis_name="s")
    spmm = pl.kernel(
        functools.partial(_spmm_sc_body, n, ep),
        out_type=jax.ShapeDtypeStruct((2, n, DB), f32),
        mesh=mesh,
        scratch_types=[
            pltpu.VMEM((IR, CHUNK), jnp.int32),
            pltpu.VMEM((IR, CHUNK), jnp.int32),
            pltpu.VMEM((IR, CHUNK), f32),
            pltpu.VMEM((GR, CHUNK, DB), f32),
            pltpu.VMEM_SHARED((n, DB), f32),
            pltpu.VMEM_SHARED((n, DB), f32),
            pltpu.SemaphoreType.DMA,
            pltpu.SemaphoreType.DMA,
            pltpu.SemaphoreType.DMA,
            pltpu.SemaphoreType.DMA,
        ],
        compiler_params=pltpu.CompilerParams(use_tc_tiling_on_sc=False),
    )
    seg = spmm(xpn, pos_index, pos_weight, neg_index, neg_weight, zeros)

    gs = jnp.stack([g_org, g_pos, g_neg]).reshape(3, DB)
    bs = jnp.stack([b_org, b_pos, b_neg]).reshape(3, DB)
    embs = pl.pallas_call(
        _post_body,
        out_shape=jax.ShapeDtypeStruct((n, DB), f32),
        in_specs=[
            pl.BlockSpec(memory_space=pltpu.VMEM),
            pl.BlockSpec(memory_space=pltpu.VMEM),
            pl.BlockSpec(memory_space=pltpu.VMEM),
            pl.BlockSpec(memory_space=pltpu.VMEM),
            pl.BlockSpec(memory_space=pltpu.VMEM),
            pl.BlockSpec(memory_space=pltpu.VMEM),
            pl.BlockSpec(memory_space=pltpu.SMEM),
        ],
    )(xo, seg[0], seg[1], W_mlp, gs, bs, prelu_a.reshape(1))
    return embs


# speedup vs baseline: 1.8951x; 1.1411x over previous
import functools

import jax
import jax.numpy as jnp
from jax import lax
from jax.experimental import pallas as pl
from jax.experimental.pallas import tpu as pltpu
from jax.experimental.pallas import tpu_sc as plsc

DB = 32
NS = 16
LANES = 16
CHUNK = 1024
GR = 2
IR = 4


def _vbroadcast(vec, lane):
    idx = jnp.full((LANES, 1), lane, jnp.int32)
    return lax.gather(
        vec, idx,
        lax.GatherDimensionNumbers(offset_dims=(), collapsed_slice_dims=(0,),
                                   start_index_map=(0,)),
        (1,), mode=lax.GatherScatterMode.PROMISE_IN_BOUNDS)


def _proj_body(x_ref, w_ref, o_ref):
    o_ref[...] = jnp.dot(x_ref[...], w_ref[...],
                         preferred_element_type=jnp.float32)


def _post_body(cat_ref, seg_ref, wm1_ref, wm2_ref, g1_ref, b1_ref, g2_ref,
               b2_ref, a_ref, o_ref):
    n = cat_ref.shape[0]
    a = a_ref[0]

    def bn_prelu(v, g, b):
        mean = jnp.sum(v, axis=0, keepdims=True) * (1.0 / n)
        var = jnp.sum(v * v, axis=0, keepdims=True) * (1.0 / n) - mean * mean
        y = g * (v - mean) * jax.lax.rsqrt(var + 1e-5) + b
        return jnp.where(y >= 0, y, a * y)

    y1 = bn_prelu(cat_ref[:, 0:DB], g1_ref[...], b1_ref[...])
    y2 = bn_prelu(seg_ref[:, 0:2 * DB], g2_ref[...], b2_ref[...])
    e = (jnp.dot(y1, wm1_ref[...], preferred_element_type=jnp.float32) +
         jnp.dot(y2, wm2_ref[...], preferred_element_type=jnp.float32))
    nrm = jnp.sqrt(jnp.sum(e * e, axis=1, keepdims=True))
    o_ref[...] = e / jnp.maximum(nrm, 1e-12)


def _spmm_sc_body(n, ep, cat_hbm, pidx_hbm, pw_hbm, nidx_hbm, nw_hbm, z_hbm,
                  out_hbm, col_v, row_v, w_v, gath_v, acc_s, xs_s,
                  gsem, ssem, isem_a, isem_b):
    c = lax.axis_index("c")
    s = lax.axis_index("s")

    zr = n // NS
    pltpu.sync_copy(z_hbm.at[pl.ds(s * zr, zr)], acc_s.at[pl.ds(s * zr, zr)])
    pltpu.sync_copy(cat_hbm.at[pl.ds(s * zr, zr), pl.ds(DB * (c + 1), DB)],
                    xs_s.at[pl.ds(s * zr, zr)])
    plsc.subcore_barrier()

    tile_edges = ep // NS
    n_chunks = tile_edges // CHUNK

    def fire_idx(k, sem):
        m = k % IR
        eoff = s * tile_edges + k * CHUNK

        @pl.when(c == 0)
        def _():
            pltpu.async_copy(pidx_hbm.at[1, pl.ds(eoff, CHUNK)],
                             col_v.at[m], sem)
            pltpu.async_copy(pidx_hbm.at[0, pl.ds(eoff, CHUNK)],
                             row_v.at[m], sem)
            pltpu.async_copy(pw_hbm.at[pl.ds(eoff, CHUNK)], w_v.at[m], sem)

        @pl.when(c == 1)
        def _():
            pltpu.async_copy(nidx_hbm.at[1, pl.ds(eoff, CHUNK)],
                             col_v.at[m], sem)
            pltpu.async_copy(nidx_hbm.at[0, pl.ds(eoff, CHUNK)],
                             row_v.at[m], sem)
            pltpu.async_copy(nw_hbm.at[pl.ds(eoff, CHUNK)], w_v.at[m], sem)

    def drain_idx(k, sem):
        m = k % IR
        pltpu.make_async_copy(pidx_hbm.at[1, pl.ds(0, CHUNK)],
                              col_v.at[m], sem).wait()
        pltpu.make_async_copy(pidx_hbm.at[0, pl.ds(0, CHUNK)],
                              row_v.at[m], sem).wait()
        pltpu.make_async_copy(pw_hbm.at[pl.ds(0, CHUNK)], w_v.at[m],
                              sem).wait()

    def fire_idx_alt(k):
        @pl.when(k % 2 == 0)
        def _():
            fire_idx(k, isem_a)

        @pl.when(k % 2 == 1)
        def _():
            fire_idx(k, isem_b)

    def drain_idx_alt(k):
        @pl.when(k % 2 == 0)
        def _():
            drain_idx(k, isem_a)

        @pl.when(k % 2 == 1)
        def _():
            drain_idx(k, isem_b)

    def fire_gather(k):
        pltpu.async_copy(xs_s.at[col_v.at[k % IR]], gath_v.at[k % GR], gsem)

    def drain_gather(k):
        pltpu.make_async_copy(xs_s.at[col_v.at[k % IR]], gath_v.at[k % GR],
                              gsem).wait()

    def fire_scatter(k):
        pltpu.async_copy(gath_v.at[k % GR], acc_s.at[row_v.at[k % IR]],
                         ssem, add=True)

    def drain_scatter(k):
        pltpu.make_async_copy(gath_v.at[k % GR], acc_s.at[row_v.at[k % IR]],
                              ssem).wait()

    fire_idx(0, isem_a)
    drain_idx(0, isem_a)
    fire_idx(1, isem_b)
    fire_gather(0)

    def chunk_body(k, carry):
        @pl.when(k < n_chunks - 2)
        def _():
            fire_idx_alt(k + 2)

        drain_gather(k)

        @pl.when(k >= 1)
        def _():
            drain_scatter(k - 1)

        @pl.when(k < n_chunks - 1)
        def _():
            drain_idx_alt(k + 1)
            fire_gather(k + 1)

        mg = k % GR
        mi = k % IR

        def scale_body(g, carry2):
            wgrp = w_v[mi, pl.ds(g * LANES, LANES)]
            for e in range(LANES):
                ws = _vbroadcast(wgrp, e)
                r = g * LANES + e
                gath_v[mg, r, 0:16] = gath_v[mg, r, 0:16] * ws
                gath_v[mg, r, 16:32] = gath_v[mg, r, 16:32] * ws
            return carry2

        lax.fori_loop(0, CHUNK // LANES, scale_body, 0)

        fire_scatter(k)
        return carry

    lax.fori_loop(0, n_chunks, chunk_body, 0)
    drain_scatter(n_chunks - 1)
    plsc.subcore_barrier()

    pltpu.sync_copy(acc_s.at[pl.ds(s * zr, zr)],
                    out_hbm.at[pl.ds(s * zr, zr), pl.ds(DB * c, DB)])


def kernel(x, pos_index, pos_weight, neg_index, neg_weight, other_index,
           other_weight, W_org, W_pos, W_neg, W_mlp, g_org, b_org, g_pos,
           b_pos, g_neg, b_neg, prelu_a):
    n, da = x.shape
    e = pos_index.shape[1]
    f32 = jnp.float32

    wcat = jnp.concatenate(
        [W_org, W_pos, W_neg, jnp.zeros((da, 128 - 3 * DB), f32)], axis=1)
    nblk = 10
    cat = pl.pallas_call(
        _proj_body,
        grid=(nblk,),
        in_specs=[pl.BlockSpec((n // nblk, da), lambda i: (i, 0)),
                  pl.BlockSpec((da, 128), lambda i: (0, 0))],
        out_specs=pl.BlockSpec((n // nblk, 128), lambda i: (i, 0)),
        out_shape=jax.ShapeDtypeStruct((n, 128), f32),
    )(x, wcat)

    step = NS * CHUNK
    ep = ((e + step - 1) // step) * step
    pad = ep - e
    if pad:
        pos_index = jnp.pad(pos_index, ((0, 0), (0, pad)))
        neg_index = jnp.pad(neg_index, ((0, 0), (0, pad)))
        pos_weight = jnp.pad(pos_weight, (0, pad))
        neg_weight = jnp.pad(neg_weight, (0, pad))
    zeros = jnp.zeros((n, DB), f32)

    mesh = plsc.VectorSubcoreMesh(core_axis_name="c", subcore_axis_name="s")
    spmm = pl.kernel(
        functools.partial(_spmm_sc_body, n, ep),
        out_type=jax.ShapeDtypeStruct((n, 128), f32),
        mesh=mesh,
        scratch_types=[
            pltpu.VMEM((IR, CHUNK), jnp.int32),
            pltpu.VMEM((IR, CHUNK), jnp.int32),
            pltpu.VMEM((IR, CHUNK), f32),
            pltpu.VMEM((GR, CHUNK, DB), f32),
            pltpu.VMEM_SHARED((n, DB), f32),
            pltpu.VMEM_SHARED((n, DB), f32),
            pltpu.SemaphoreType.DMA,
            pltpu.SemaphoreType.DMA,
            pltpu.SemaphoreType.DMA,
            pltpu.SemaphoreType.DMA,
        ],
        compiler_params=pltpu.CompilerParams(use_tc_tiling_on_sc=False),
    )
    seg = spmm(cat, pos_index, pos_weight, neg_index, neg_weight, zeros)

    g1 = g_org.reshape(1, DB)
    b1 = b_org.reshape(1, DB)
    g2 = jnp.concatenate([g_pos, g_neg]).reshape(1, 2 * DB)
    b2 = jnp.concatenate([b_pos, b_neg]).reshape(1, 2 * DB)
    embs = pl.pallas_call(
        _post_body,
        out_shape=jax.ShapeDtypeStruct((n, DB), f32),
        in_specs=[
            pl.BlockSpec(memory_space=pltpu.VMEM),
            pl.BlockSpec(memory_space=pltpu.VMEM),
            pl.BlockSpec(memory_space=pltpu.VMEM),
            pl.BlockSpec(memory_space=pltpu.VMEM),
            pl.BlockSpec(memory_space=pltpu.VMEM),
            pl.BlockSpec(memory_space=pltpu.VMEM),
            pl.BlockSpec(memory_space=pltpu.VMEM),
            pl.BlockSpec(memory_space=pltpu.VMEM),
            pl.BlockSpec(memory_space=pltpu.SMEM),
        ],
    )(cat, seg, W_mlp[0:DB], W_mlp[DB:3 * DB], g1, b1, g2, b2,
      prelu_a.reshape(1))
    return embs


# speedup vs baseline: 1.9089x; 1.0073x over previous
import functools

import jax
import jax.numpy as jnp
from jax import lax
from jax.experimental import pallas as pl
from jax.experimental.pallas import tpu as pltpu
from jax.experimental.pallas import tpu_sc as plsc

DB = 32
NS = 16
LANES = 16
CHUNK = 1024
GR = 2
IR = 4


def _vbroadcast(vec, lane):
    idx = jnp.full((LANES, 1), lane, jnp.int32)
    return lax.gather(
        vec, idx,
        lax.GatherDimensionNumbers(offset_dims=(), collapsed_slice_dims=(0,),
                                   start_index_map=(0,)),
        (1,), mode=lax.GatherScatterMode.PROMISE_IN_BOUNDS)


def _proj_body(x_ref, w_ref, o_ref):
    o_ref[...] = jnp.dot(x_ref[...], w_ref[...],
                         preferred_element_type=jnp.float32)


def _bn_prelu(v, g, b, a, n):
    mean = jnp.sum(v, axis=0, keepdims=True) * (1.0 / n)
    var = jnp.sum(v * v, axis=0, keepdims=True) * (1.0 / n) - mean * mean
    y = g * (v - mean) * jax.lax.rsqrt(var + 1e-5) + b
    return jnp.where(y >= 0, y, a * y)


def _post_org_body(cat_ref, wm1_ref, g1_ref, b1_ref, a_ref, e1_ref):
    n = cat_ref.shape[0]
    y1 = _bn_prelu(cat_ref[:, 0:DB], g1_ref[...], b1_ref[...], a_ref[0], n)
    e1_ref[...] = jnp.dot(y1, wm1_ref[...], preferred_element_type=jnp.float32)


def _post_body(seg_ref, e1_ref, wm2_ref, g2_ref, b2_ref, a_ref, o_ref):
    n = seg_ref.shape[0]
    y2 = _bn_prelu(seg_ref[:, 0:2 * DB], g2_ref[...], b2_ref[...], a_ref[0], n)
    e = e1_ref[...] + jnp.dot(y2, wm2_ref[...],
                              preferred_element_type=jnp.float32)
    nrm = jnp.sqrt(jnp.sum(e * e, axis=1, keepdims=True))
    o_ref[...] = e / jnp.maximum(nrm, 1e-12)


def _spmm_sc_body(n, ep, cat_hbm, pidx_hbm, pw_hbm, nidx_hbm, nw_hbm, z_hbm,
                  out_hbm, col_v, row_v, w_v, gath_v, acc_s, xs_s,
                  gsem, ssem, isem_a, isem_b):
    c = lax.axis_index("c")
    s = lax.axis_index("s")

    zr = n // NS
    pltpu.sync_copy(z_hbm.at[pl.ds(s * zr, zr)], acc_s.at[pl.ds(s * zr, zr)])
    pltpu.sync_copy(cat_hbm.at[pl.ds(s * zr, zr), pl.ds(DB * (c + 1), DB)],
                    xs_s.at[pl.ds(s * zr, zr)])
    plsc.subcore_barrier()

    tile_edges = ep // NS
    n_chunks = tile_edges // CHUNK

    def fire_idx(k, sem):
        m = k % IR
        eoff = s * tile_edges + k * CHUNK

        @pl.when(c == 0)
        def _():
            pltpu.async_copy(pidx_hbm.at[1, pl.ds(eoff, CHUNK)],
                             col_v.at[m], sem)
            pltpu.async_copy(pidx_hbm.at[0, pl.ds(eoff, CHUNK)],
                             row_v.at[m], sem)
            pltpu.async_copy(pw_hbm.at[pl.ds(eoff, CHUNK)], w_v.at[m], sem)

        @pl.when(c == 1)
        def _():
            pltpu.async_copy(nidx_hbm.at[1, pl.ds(eoff, CHUNK)],
                             col_v.at[m], sem)
            pltpu.async_copy(nidx_hbm.at[0, pl.ds(eoff, CHUNK)],
                             row_v.at[m], sem)
            pltpu.async_copy(nw_hbm.at[pl.ds(eoff, CHUNK)], w_v.at[m], sem)

    def drain_idx(k, sem):
        m = k % IR
        pltpu.make_async_copy(pidx_hbm.at[1, pl.ds(0, CHUNK)],
                              col_v.at[m], sem).wait()
        pltpu.make_async_copy(pidx_hbm.at[0, pl.ds(0, CHUNK)],
                              row_v.at[m], sem).wait()
        pltpu.make_async_copy(pw_hbm.at[pl.ds(0, CHUNK)], w_v.at[m],
                              sem).wait()

    def fire_idx_alt(k):
        @pl.when(k % 2 == 0)
        def _():
            fire_idx(k, isem_a)

        @pl.when(k % 2 == 1)
        def _():
            fire_idx(k, isem_b)

    def drain_idx_alt(k):
        @pl.when(k % 2 == 0)
        def _():
            drain_idx(k, isem_a)

        @pl.when(k % 2 == 1)
        def _():
            drain_idx(k, isem_b)

    def fire_gather(k):
        pltpu.async_copy(xs_s.at[col_v.at[k % IR]], gath_v.at[k % GR], gsem)

    def drain_gather(k):
        pltpu.make_async_copy(xs_s.at[col_v.at[k % IR]], gath_v.at[k % GR],
                              gsem).wait()

    def fire_scatter(k):
        pltpu.async_copy(gath_v.at[k % GR], acc_s.at[row_v.at[k % IR]],
                         ssem, add=True)

    def drain_scatter(k):
        pltpu.make_async_copy(gath_v.at[k % GR], acc_s.at[row_v.at[k % IR]],
                              ssem).wait()

    fire_idx(0, isem_a)
    drain_idx(0, isem_a)
    fire_idx(1, isem_b)
    fire_gather(0)

    def chunk_body(k, carry):
        @pl.when(k < n_chunks - 2)
        def _():
            fire_idx_alt(k + 2)

        drain_gather(k)

        @pl.when(k >= 1)
        def _():
            drain_scatter(k - 1)

        @pl.when(k < n_chunks - 1)
        def _():
            drain_idx_alt(k + 1)
            fire_gather(k + 1)

        mg = k % GR
        mi = k % IR

        def scale_body(g, carry2):
            wgrp = w_v[mi, pl.ds(g * LANES, LANES)]
            for e in range(LANES):
                ws = _vbroadcast(wgrp, e)
                r = g * LANES + e
                gath_v[mg, r, 0:16] = gath_v[mg, r, 0:16] * ws
                gath_v[mg, r, 16:32] = gath_v[mg, r, 16:32] * ws
            return carry2

        lax.fori_loop(0, CHUNK // LANES, scale_body, 0)

        fire_scatter(k)
        return carry

    lax.fori_loop(0, n_chunks, chunk_body, 0)
    drain_scatter(n_chunks - 1)
    plsc.subcore_barrier()

    pltpu.sync_copy(acc_s.at[pl.ds(s * zr, zr)],
                    out_hbm.at[pl.ds(s * zr, zr), pl.ds(DB * c, DB)])


def kernel(x, pos_index, pos_weight, neg_index, neg_weight, other_index,
           other_weight, W_org, W_pos, W_neg, W_mlp, g_org, b_org, g_pos,
           b_pos, g_neg, b_neg, prelu_a):
    n, da = x.shape
    e = pos_index.shape[1]
    f32 = jnp.float32

    wcat = jnp.concatenate(
        [W_org, W_pos, W_neg, jnp.zeros((da, 128 - 3 * DB), f32)], axis=1)
    nblk = 10
    cat = pl.pallas_call(
        _proj_body,
        grid=(nblk,),
        in_specs=[pl.BlockSpec((n // nblk, da), lambda i: (i, 0)),
                  pl.BlockSpec((da, 128), lambda i: (0, 0))],
        out_specs=pl.BlockSpec((n // nblk, 128), lambda i: (i, 0)),
        out_shape=jax.ShapeDtypeStruct((n, 128), f32),
    )(x, wcat)

    step = NS * CHUNK
    ep = ((e + step - 1) // step) * step
    pad = ep - e
    if pad:
        pos_index = jnp.pad(pos_index, ((0, 0), (0, pad)))
        neg_index = jnp.pad(neg_index, ((0, 0), (0, pad)))
        pos_weight = jnp.pad(pos_weight, (0, pad))
        neg_weight = jnp.pad(neg_weight, (0, pad))
    zeros = jnp.zeros((n, DB), f32)

    mesh = plsc.VectorSubcoreMesh(core_axis_name="c", subcore_axis_name="s")
    spmm = pl.kernel(
        functools.partial(_spmm_sc_body, n, ep),
        out_type=jax.ShapeDtypeStruct((n, 128), f32),
        mesh=mesh,
        scratch_types=[
            pltpu.VMEM((IR, CHUNK), jnp.int32),
            pltpu.VMEM((IR, CHUNK), jnp.int32),
            pltpu.VMEM((IR, CHUNK), f32),
            pltpu.VMEM((GR, CHUNK, DB), f32),
            pltpu.VMEM_SHARED((n, DB), f32),
            pltpu.VMEM_SHARED((n, DB), f32),
            pltpu.SemaphoreType.DMA,
            pltpu.SemaphoreType.DMA,
            pltpu.SemaphoreType.DMA,
            pltpu.SemaphoreType.DMA,
        ],
        compiler_params=pltpu.CompilerParams(use_tc_tiling_on_sc=False),
    )
    seg = spmm(cat, pos_index, pos_weight, neg_index, neg_weight, zeros)

    g1 = g_org.reshape(1, DB)
    b1 = b_org.reshape(1, DB)
    g2 = jnp.concatenate([g_pos, g_neg]).reshape(1, 2 * DB)
    b2 = jnp.concatenate([b_pos, b_neg]).reshape(1, 2 * DB)
    ap = prelu_a.reshape(1)
    vm = pl.BlockSpec(memory_space=pltpu.VMEM)
    sm = pl.BlockSpec(memory_space=pltpu.SMEM)
    e1 = pl.pallas_call(
        _post_org_body,
        out_shape=jax.ShapeDtypeStruct((n, DB), f32),
        in_specs=[vm, vm, vm, vm, sm],
    )(cat, W_mlp[0:DB], g1, b1, ap)
    embs = pl.pallas_call(
        _post_body,
        out_shape=jax.ShapeDtypeStruct((n, DB), f32),
        in_specs=[vm, vm, vm, vm, vm, sm],
    )(seg, e1, W_mlp[DB:3 * DB], g2, b2, ap)
    return embs
